# baseline (speedup 1.0000x reference)
import functools

import jax
import jax.numpy as jnp
from jax import lax
from jax.experimental import pallas as pl
from jax.experimental.pallas import tpu as pltpu
from jax.experimental.pallas import tpu_sc as plsc

N_NODES = 100000
N_PAD = 102400
EMB_DIM = 128
MAX_DEG = 256
BLK = 2048

E_TOTAL = 1600000
E_PAD = 1638400
CHUNK = 2048
E_PER_SUB = E_PAD // 16
NCHUNKS = E_PER_SUB // CHUNK
NPAIRS = NCHUNKS // 2
SEG = N_PAD // 16


_sc_mesh = plsc.VectorSubcoreMesh(core_axis_name="c", subcore_axis_name="s")


@functools.partial(
    pl.kernel,
    out_type=jax.ShapeDtypeStruct((2, N_PAD), jnp.int32),
    mesh=_sc_mesh,
    scratch_types=[
        pltpu.VMEM_SHARED((N_PAD,), jnp.int32),
        pltpu.VMEM((2, CHUNK), jnp.int32),
        pltpu.VMEM((CHUNK,), jnp.int32),
        pltpu.VMEM((CHUNK,), jnp.int32),
        pltpu.VMEM((2, CHUNK), jnp.int32),
        pltpu.VMEM((CHUNK,), jnp.int32),
        pltpu.VMEM((CHUNK,), jnp.int32),
        pltpu.SemaphoreType.DMA,
        pltpu.SemaphoreType.DMA,
    ],
)
def _sc_degrees(edgesH, outH, acc,
                ebuf0, idx0, val0, ebuf1, idx1, val1, sem0, sem1):
    c = lax.axis_index("c")
    s = lax.axis_index("s")
    bufs = ((ebuf0, idx0, val0, sem0), (ebuf1, idx1, val1, sem1))

    def load_pair(chunk, ebuf, sem):
        e0 = s * E_PER_SUB + chunk * CHUNK
        pltpu.async_copy(edgesH.at[:, pl.ds(e0, CHUNK)], ebuf, sem)

    def wait_pair(ebuf, sem):
        pltpu.make_async_copy(edgesH.at[:, pl.ds(0, CHUNK)], ebuf, sem).wait()

    load_pair(0, ebuf0, sem0)
    load_pair(1, ebuf1, sem1)

    def zero_body(i, carry):
        val0[pl.ds(i * 16, 16)] = jnp.zeros((16,), jnp.int32)
        return carry

    lax.fori_loop(0, CHUNK // 16, zero_body, 0)
    base = s * SEG
    for k in range(3):
        pltpu.sync_copy(val0, acc.at[pl.ds(base + k * CHUNK, CHUNK)])
    pltpu.sync_copy(val0.at[pl.ds(0, 256)], acc.at[pl.ds(base + 3 * CHUNK, 256)])
    plsc.subcore_barrier()

    def pair_body(t, carry):
        for b in range(2):
            ebuf, ibuf, vbuf, sem = bufs[b]
            chunk = t * 2 + b
            wait_pair(ebuf, sem)

            def vec_body(i, carry2):
                for u in range(4):
                    o = i * 64 + u * 16
                    sv = ebuf[0, pl.ds(o, 16)]
                    dv = ebuf[1, pl.ds(o, 16)]
                    vbuf[pl.ds(o, 16)] = jnp.where(
                        sv != dv, jnp.int32(1), jnp.int32(0))
                    ibuf[pl.ds(o, 16)] = jnp.where(c == 0, dv, sv)
                return carry2

            lax.fori_loop(0, CHUNK // 64, vec_body, 0)

            pltpu.sync_copy(vbuf, acc.at[ibuf], add=True)

            @pl.when(chunk + 2 < NCHUNKS)
            def _():
                load_pair(chunk + 2, ebuf, sem)
        return carry

    lax.fori_loop(0, NPAIRS, pair_body, 0)
    plsc.subcore_barrier()
    pltpu.sync_copy(acc.at[pl.ds(base, SEG)], outH.at[c, pl.ds(base, SEG)])


def _tc_body(x_ref, wt_ref, b_ref, ind_ref, outd_ref, ie_ref, oe_ref, o_ref):
    h = jnp.dot(x_ref[...], wt_ref[...], preferred_element_type=jnp.float32)
    h = h + b_ref[...]
    iota = lax.broadcasted_iota(jnp.int32, (MAX_DEG, BLK), 0)
    ind = jnp.clip(ind_ref[0], 0, MAX_DEG - 1)
    outd = jnp.clip(outd_ref[0], 0, MAX_DEG - 1)
    dn = (((0,), (0,)), ((), ()))
    oh_in = (ind == iota).astype(jnp.bfloat16)
    oh_out = (outd == iota).astype(jnp.bfloat16)
    h = h + lax.dot_general(oh_in, ie_ref[...], dn,
                            preferred_element_type=jnp.float32)
    h = h + lax.dot_general(oh_out, oe_ref[...], dn,
                            preferred_element_type=jnp.float32)
    o_ref[...] = h


def _tc_pass(x, Wt, b2, deg, in_emb, out_emb):
    grid = (pl.cdiv(N_NODES, BLK),)
    return pl.pallas_call(
        _tc_body,
        grid=grid,
        in_specs=[
            pl.BlockSpec((BLK, x.shape[1]), lambda i: (i, 0)),
            pl.BlockSpec(Wt.shape, lambda i: (0, 0)),
            pl.BlockSpec(b2.shape, lambda i: (0, 0)),
            pl.BlockSpec((1, 1, BLK), lambda i: (0, 0, i)),
            pl.BlockSpec((1, 1, BLK), lambda i: (1, 0, i)),
            pl.BlockSpec(in_emb.shape, lambda i: (0, 0)),
            pl.BlockSpec(out_emb.shape, lambda i: (0, 0)),
        ],
        out_specs=pl.BlockSpec((BLK, EMB_DIM), lambda i: (i, 0)),
        out_shape=jax.ShapeDtypeStruct((N_NODES, EMB_DIM), jnp.float32),
    )(x, Wt, b2, deg, deg, in_emb, out_emb)


def kernel(x, edge_index, W, b, in_emb, out_emb):
    ei = edge_index.astype(jnp.int32)
    ei = jnp.pad(ei, ((0, 0), (0, E_PAD - E_TOTAL)))
    deg = _sc_degrees(ei).reshape(2, 1, N_PAD)
    return _tc_pass(x, W.T, b.reshape(1, EMB_DIM), deg,
                    in_emb.astype(jnp.bfloat16), out_emb.astype(jnp.bfloat16))


# speedup vs baseline: 1.1350x; 1.1350x over previous
import functools

import jax
import jax.numpy as jnp
from jax import lax
from jax.experimental import pallas as pl
from jax.experimental.pallas import tpu as pltpu
from jax.experimental.pallas import tpu_sc as plsc

N_NODES = 100000
N_PAD = 102400
EMB_DIM = 128
MAX_DEG = 256
BLK = 2048

E_TOTAL = 1600000
E_PAD = 1638400
CHUNK = 2048
E_PER_SUB = E_PAD // 16
NCHUNKS = E_PER_SUB // CHUNK
NPAIRS = NCHUNKS // 2
SEG = N_PAD // 16


_sc_mesh = plsc.VectorSubcoreMesh(core_axis_name="c", subcore_axis_name="s")


@functools.partial(
    pl.kernel,
    out_type=jax.ShapeDtypeStruct((2, N_PAD), jnp.int32),
    mesh=_sc_mesh,
    scratch_types=[
        pltpu.VMEM_SHARED((N_PAD,), jnp.int32),
        pltpu.VMEM((2, CHUNK), jnp.int32),
        pltpu.VMEM((CHUNK,), jnp.int32),
        pltpu.VMEM((CHUNK,), jnp.int32),
        pltpu.VMEM((2, CHUNK), jnp.int32),
        pltpu.VMEM((CHUNK,), jnp.int32),
        pltpu.VMEM((CHUNK,), jnp.int32),
        pltpu.SemaphoreType.DMA,
        pltpu.SemaphoreType.DMA,
        pltpu.SemaphoreType.DMA,
        pltpu.SemaphoreType.DMA,
    ],
)
def _sc_degrees(edgesH, outH, acc,
                ebuf0, idx0, val0, ebuf1, idx1, val1,
                sem0, sem1, ssem0, ssem1):
    c = lax.axis_index("c")
    s = lax.axis_index("s")
    bufs = ((ebuf0, idx0, val0, sem0, ssem0), (ebuf1, idx1, val1, sem1, ssem1))

    def load_pair(chunk, ebuf, sem):
        e0 = s * E_PER_SUB + chunk * CHUNK
        pltpu.async_copy(edgesH.at[:, pl.ds(e0, CHUNK)], ebuf, sem)

    def wait_pair(ebuf, sem):
        pltpu.make_async_copy(edgesH.at[:, pl.ds(0, CHUNK)], ebuf, sem).wait()

    load_pair(0, ebuf0, sem0)
    load_pair(1, ebuf1, sem1)

    def zero_body(i, carry):
        val0[pl.ds(i * 16, 16)] = jnp.zeros((16,), jnp.int32)
        return carry

    lax.fori_loop(0, CHUNK // 16, zero_body, 0)
    base = s * SEG
    for k in range(3):
        pltpu.sync_copy(val0, acc.at[pl.ds(base + k * CHUNK, CHUNK)])
    pltpu.sync_copy(val0.at[pl.ds(0, 256)], acc.at[pl.ds(base + 3 * CHUNK, 256)])
    plsc.subcore_barrier()

    def pair_body(t, carry):
        for b in range(2):
            ebuf, ibuf, vbuf, sem, ssem = bufs[b]
            chunk = t * 2 + b
            wait_pair(ebuf, sem)

            @pl.when(t > 0)
            def _():
                pltpu.make_async_copy(vbuf, acc.at[ibuf], ssem).wait()

            def vec_body(i, carry2):
                for u in range(4):
                    o = i * 64 + u * 16
                    sv = ebuf[0, pl.ds(o, 16)]
                    dv = ebuf[1, pl.ds(o, 16)]
                    vbuf[pl.ds(o, 16)] = jnp.where(
                        sv != dv, jnp.int32(1), jnp.int32(0))
                    ibuf[pl.ds(o, 16)] = jnp.where(c == 0, dv, sv)
                return carry2

            lax.fori_loop(0, CHUNK // 64, vec_body, 0)

            pltpu.make_async_copy(vbuf, acc.at[ibuf], ssem).start(add=True)

            @pl.when(chunk + 2 < NCHUNKS)
            def _():
                load_pair(chunk + 2, ebuf, sem)
        return carry

    lax.fori_loop(0, NPAIRS, pair_body, 0)
    for b in range(2):
        ebuf, ibuf, vbuf, sem, ssem = bufs[b]
        pltpu.make_async_copy(vbuf, acc.at[ibuf], ssem).wait()
    plsc.subcore_barrier()
    pltpu.sync_copy(acc.at[pl.ds(base, SEG)], outH.at[c, pl.ds(base, SEG)])


def _tc_body(x_ref, wt_ref, b_ref, ind_ref, outd_ref, ie_ref, oe_ref, o_ref):
    h = jnp.dot(x_ref[...], wt_ref[...], preferred_element_type=jnp.float32)
    h = h + b_ref[...]
    iota = lax.broadcasted_iota(jnp.int32, (MAX_DEG, BLK), 0)
    ind = jnp.clip(ind_ref[0], 0, MAX_DEG - 1)
    outd = jnp.clip(outd_ref[0], 0, MAX_DEG - 1)
    dn = (((0,), (0,)), ((), ()))
    oh_in = (ind == iota).astype(jnp.bfloat16)
    oh_out = (outd == iota).astype(jnp.bfloat16)
    h = h + lax.dot_general(oh_in, ie_ref[...], dn,
                            preferred_element_type=jnp.float32)
    h = h + lax.dot_general(oh_out, oe_ref[...], dn,
                            preferred_element_type=jnp.float32)
    o_ref[...] = h


def _tc_pass(x, Wt, b2, deg, in_emb, out_emb):
    grid = (pl.cdiv(N_NODES, BLK),)
    return pl.pallas_call(
        _tc_body,
        grid=grid,
        in_specs=[
            pl.BlockSpec((BLK, x.shape[1]), lambda i: (i, 0)),
            pl.BlockSpec(Wt.shape, lambda i: (0, 0)),
            pl.BlockSpec(b2.shape, lambda i: (0, 0)),
            pl.BlockSpec((1, 1, BLK), lambda i: (0, 0, i)),
            pl.BlockSpec((1, 1, BLK), lambda i: (1, 0, i)),
            pl.BlockSpec(in_emb.shape, lambda i: (0, 0)),
            pl.BlockSpec(out_emb.shape, lambda i: (0, 0)),
        ],
        out_specs=pl.BlockSpec((BLK, EMB_DIM), lambda i: (i, 0)),
        out_shape=jax.ShapeDtypeStruct((N_NODES, EMB_DIM), jnp.float32),
    )(x, Wt, b2, deg, deg, in_emb, out_emb)


def kernel(x, edge_index, W, b, in_emb, out_emb):
    ei = edge_index.astype(jnp.int32)
    ei = jnp.pad(ei, ((0, 0), (0, E_PAD - E_TOTAL)))
    deg = _sc_degrees(ei).reshape(2, 1, N_PAD)
    return _tc_pass(x, W.T, b.reshape(1, EMB_DIM), deg,
                    in_emb.astype(jnp.bfloat16), out_emb.astype(jnp.bfloat16))


# speedup vs baseline: 1.2940x; 1.1401x over previous
import functools

import jax
import jax.numpy as jnp
from jax import lax
from jax.experimental import pallas as pl
from jax.experimental.pallas import tpu as pltpu
from jax.experimental.pallas import tpu_sc as plsc

N_NODES = 100000
N_PAD = 102400
EMB_DIM = 128
MAX_DEG = 256
BLK = 6400

E_TOTAL = 1600000
E_PAD = 1638400
CHUNK = 2048
E_PER_SUB = E_PAD // 16
NCHUNKS = E_PER_SUB // CHUNK
NPAIRS = NCHUNKS // 2
SEG = N_PAD // 16


_sc_mesh = plsc.VectorSubcoreMesh(core_axis_name="c", subcore_axis_name="s")


@functools.partial(
    pl.kernel,
    out_type=jax.ShapeDtypeStruct((2, N_PAD), jnp.int32),
    mesh=_sc_mesh,
    scratch_types=[
        pltpu.VMEM_SHARED((N_PAD,), jnp.int32),
        pltpu.VMEM((2, CHUNK), jnp.int32),
        pltpu.VMEM((CHUNK,), jnp.int32),
        pltpu.VMEM((CHUNK,), jnp.int32),
        pltpu.VMEM((2, CHUNK), jnp.int32),
        pltpu.VMEM((CHUNK,), jnp.int32),
        pltpu.VMEM((CHUNK,), jnp.int32),
        pltpu.SemaphoreType.DMA,
        pltpu.SemaphoreType.DMA,
        pltpu.SemaphoreType.DMA,
        pltpu.SemaphoreType.DMA,
    ],
)
def _sc_degrees(edgesH, outH, acc,
                ebuf0, idx0, val0, ebuf1, idx1, val1,
                sem0, sem1, ssem0, ssem1):
    c = lax.axis_index("c")
    s = lax.axis_index("s")
    bufs = ((ebuf0, idx0, val0, sem0, ssem0), (ebuf1, idx1, val1, sem1, ssem1))

    def load_pair(chunk, ebuf, sem):
        e0 = s * E_PER_SUB + chunk * CHUNK
        pltpu.async_copy(edgesH.at[:, pl.ds(e0, CHUNK)], ebuf, sem)

    def wait_pair(ebuf, sem):
        pltpu.make_async_copy(edgesH.at[:, pl.ds(0, CHUNK)], ebuf, sem).wait()

    load_pair(0, ebuf0, sem0)
    load_pair(1, ebuf1, sem1)

    def zero_body(i, carry):
        val0[pl.ds(i * 16, 16)] = jnp.zeros((16,), jnp.int32)
        return carry

    lax.fori_loop(0, CHUNK // 16, zero_body, 0)
    base = s * SEG
    for k in range(3):
        pltpu.sync_copy(val0, acc.at[pl.ds(base + k * CHUNK, CHUNK)])
    pltpu.sync_copy(val0.at[pl.ds(0, 256)], acc.at[pl.ds(base + 3 * CHUNK, 256)])
    plsc.subcore_barrier()

    def pair_body(t, carry):
        for b in range(2):
            ebuf, ibuf, vbuf, sem, ssem = bufs[b]
            chunk = t * 2 + b
            wait_pair(ebuf, sem)

            @pl.when(t > 0)
            def _():
                pltpu.make_async_copy(vbuf, acc.at[ibuf], ssem).wait()

            def vec_body(i, carry2):
                for u in range(4):
                    o = i * 64 + u * 16
                    sv = ebuf[0, pl.ds(o, 16)]
                    dv = ebuf[1, pl.ds(o, 16)]
                    vbuf[pl.ds(o, 16)] = jnp.where(
                        sv != dv, jnp.int32(1), jnp.int32(0))
                    ibuf[pl.ds(o, 16)] = jnp.where(c == 0, dv, sv)
                return carry2

            lax.fori_loop(0, CHUNK // 64, vec_body, 0)

            pltpu.make_async_copy(vbuf, acc.at[ibuf], ssem).start(add=True)

            @pl.when(chunk + 2 < NCHUNKS)
            def _():
                load_pair(chunk + 2, ebuf, sem)
        return carry

    lax.fori_loop(0, NPAIRS, pair_body, 0)
    for b in range(2):
        ebuf, ibuf, vbuf, sem, ssem = bufs[b]
        pltpu.make_async_copy(vbuf, acc.at[ibuf], ssem).wait()
    plsc.subcore_barrier()
    pltpu.sync_copy(acc.at[pl.ds(base, SEG)], outH.at[c, pl.ds(base, SEG)])


def _tc_body(x_ref, wt_ref, b_ref, ind_ref, outd_ref, ie_ref, oe_ref, o_ref):
    h = jnp.dot(x_ref[...], wt_ref[...], preferred_element_type=jnp.float32)
    h = h + b_ref[...]
    iota = lax.broadcasted_iota(jnp.int32, (MAX_DEG, BLK), 0)
    ind = jnp.clip(ind_ref[0], 0, MAX_DEG - 1)
    outd = jnp.clip(outd_ref[0], 0, MAX_DEG - 1)
    dn = (((0,), (0,)), ((), ()))
    oh_in = (ind == iota).astype(jnp.bfloat16)
    oh_out = (outd == iota).astype(jnp.bfloat16)
    h = h + lax.dot_general(oh_in, ie_ref[...], dn,
                            preferred_element_type=jnp.float32)
    h = h + lax.dot_general(oh_out, oe_ref[...], dn,
                            preferred_element_type=jnp.float32)
    o_ref[...] = h


def _tc_pass(x, Wt, b2, deg, in_emb, out_emb):
    grid = (pl.cdiv(N_NODES, BLK),)
    return pl.pallas_call(
        _tc_body,
        grid=grid,
        in_specs=[
            pl.BlockSpec((BLK, x.shape[1]), lambda i: (i, 0)),
            pl.BlockSpec(Wt.shape, lambda i: (0, 0)),
            pl.BlockSpec(b2.shape, lambda i: (0, 0)),
            pl.BlockSpec((1, 1, BLK), lambda i: (0, 0, i)),
            pl.BlockSpec((1, 1, BLK), lambda i: (1, 0, i)),
            pl.BlockSpec(in_emb.shape, lambda i: (0, 0)),
            pl.BlockSpec(out_emb.shape, lambda i: (0, 0)),
        ],
        out_specs=pl.BlockSpec((BLK, EMB_DIM), lambda i: (i, 0)),
        out_shape=jax.ShapeDtypeStruct((N_NODES, EMB_DIM), jnp.float32),
    )(x, Wt, b2, deg, deg, in_emb, out_emb)


def kernel(x, edge_index, W, b, in_emb, out_emb):
    ei = edge_index.astype(jnp.int32)
    ei = jnp.pad(ei, ((0, 0), (0, E_PAD - E_TOTAL)))
    deg = _sc_degrees(ei).reshape(2, 1, N_PAD)
    return _tc_pass(x, W.T, b.reshape(1, EMB_DIM), deg,
                    in_emb.astype(jnp.bfloat16), out_emb.astype(jnp.bfloat16))


# speedup vs baseline: 1.3239x; 1.0231x over previous
import functools

import jax
import jax.numpy as jnp
from jax import lax
from jax.experimental import pallas as pl
from jax.experimental.pallas import tpu as pltpu
from jax.experimental.pallas import tpu_sc as plsc

N_NODES = 100000
N_PAD = 102400
EMB_DIM = 128
MAX_DEG = 256
BLK = 12800

E_TOTAL = 1600000
E_PAD = 1638400
CHUNK = 2048
E_PER_SUB = E_PAD // 16
NCHUNKS = E_PER_SUB // CHUNK
NPAIRS = NCHUNKS // 2
SEG = N_PAD // 16


_sc_mesh = plsc.VectorSubcoreMesh(core_axis_name="c", subcore_axis_name="s")


@functools.partial(
    pl.kernel,
    out_type=jax.ShapeDtypeStruct((2, N_PAD), jnp.int32),
    mesh=_sc_mesh,
    scratch_types=[
        pltpu.VMEM_SHARED((N_PAD,), jnp.int32),
        pltpu.VMEM((2, CHUNK), jnp.int32),
        pltpu.VMEM((CHUNK,), jnp.int32),
        pltpu.VMEM((CHUNK,), jnp.int32),
        pltpu.VMEM((2, CHUNK), jnp.int32),
        pltpu.VMEM((CHUNK,), jnp.int32),
        pltpu.VMEM((CHUNK,), jnp.int32),
        pltpu.SemaphoreType.DMA,
        pltpu.SemaphoreType.DMA,
        pltpu.SemaphoreType.DMA,
        pltpu.SemaphoreType.DMA,
    ],
)
def _sc_degrees(edgesH, outH, acc,
                ebuf0, idx0, val0, ebuf1, idx1, val1,
                sem0, sem1, ssem0, ssem1):
    c = lax.axis_index("c")
    s = lax.axis_index("s")
    bufs = ((ebuf0, idx0, val0, sem0, ssem0), (ebuf1, idx1, val1, sem1, ssem1))

    def load_pair(chunk, ebuf, sem):
        e0 = s * E_PER_SUB + chunk * CHUNK
        pltpu.async_copy(edgesH.at[:, pl.ds(e0, CHUNK)], ebuf, sem)

    def wait_pair(ebuf, sem):
        pltpu.make_async_copy(edgesH.at[:, pl.ds(0, CHUNK)], ebuf, sem).wait()

    load_pair(0, ebuf0, sem0)
    load_pair(1, ebuf1, sem1)

    def zero_body(i, carry):
        val0[pl.ds(i * 16, 16)] = jnp.zeros((16,), jnp.int32)
        return carry

    lax.fori_loop(0, CHUNK // 16, zero_body, 0)
    base = s * SEG
    for k in range(3):
        pltpu.sync_copy(val0, acc.at[pl.ds(base + k * CHUNK, CHUNK)])
    pltpu.sync_copy(val0.at[pl.ds(0, 256)], acc.at[pl.ds(base + 3 * CHUNK, 256)])
    plsc.subcore_barrier()

    def pair_body(t, carry):
        for b in range(2):
            ebuf, ibuf, vbuf, sem, ssem = bufs[b]
            chunk = t * 2 + b
            wait_pair(ebuf, sem)

            @pl.when(t > 0)
            def _():
                pltpu.make_async_copy(vbuf, acc.at[ibuf], ssem).wait()

            def vec_body(i, carry2):
                for u in range(4):
                    o = i * 64 + u * 16
                    sv = ebuf[0, pl.ds(o, 16)]
                    dv = ebuf[1, pl.ds(o, 16)]
                    vbuf[pl.ds(o, 16)] = jnp.where(
                        sv != dv, jnp.int32(1), jnp.int32(0))
                    ibuf[pl.ds(o, 16)] = jnp.where(c == 0, dv, sv)
                return carry2

            lax.fori_loop(0, CHUNK // 64, vec_body, 0)

            pltpu.make_async_copy(vbuf, acc.at[ibuf], ssem).start(add=True)

            @pl.when(chunk + 2 < NCHUNKS)
            def _():
                load_pair(chunk + 2, ebuf, sem)
        return carry

    lax.fori_loop(0, NPAIRS, pair_body, 0)
    for b in range(2):
        ebuf, ibuf, vbuf, sem, ssem = bufs[b]
        pltpu.make_async_copy(vbuf, acc.at[ibuf], ssem).wait()
    plsc.subcore_barrier()
    pltpu.sync_copy(acc.at[pl.ds(base, SEG)], outH.at[c, pl.ds(base, SEG)])


def _tc_body(x_ref, wt_ref, b_ref, ind_ref, outd_ref, ie_ref, oe_ref, o_ref):
    h = jnp.dot(x_ref[...], wt_ref[...], preferred_element_type=jnp.float32)
    h = h + b_ref[...]
    iota = lax.broadcasted_iota(jnp.int32, (MAX_DEG, BLK), 0)
    ind = jnp.clip(ind_ref[0], 0, MAX_DEG - 1)
    outd = jnp.clip(outd_ref[0], 0, MAX_DEG - 1)
    dn = (((0,), (0,)), ((), ()))
    oh_in = (ind == iota).astype(jnp.bfloat16)
    oh_out = (outd == iota).astype(jnp.bfloat16)
    h = h + lax.dot_general(oh_in, ie_ref[...], dn,
                            preferred_element_type=jnp.float32)
    h = h + lax.dot_general(oh_out, oe_ref[...], dn,
                            preferred_element_type=jnp.float32)
    o_ref[...] = h


def _tc_pass(x, Wt, b2, deg, in_emb, out_emb):
    grid = (pl.cdiv(N_NODES, BLK),)
    return pl.pallas_call(
        _tc_body,
        grid=grid,
        in_specs=[
            pl.BlockSpec((BLK, x.shape[1]), lambda i: (i, 0)),
            pl.BlockSpec(Wt.shape, lambda i: (0, 0)),
            pl.BlockSpec(b2.shape, lambda i: (0, 0)),
            pl.BlockSpec((1, 1, BLK), lambda i: (0, 0, i)),
            pl.BlockSpec((1, 1, BLK), lambda i: (1, 0, i)),
            pl.BlockSpec(in_emb.shape, lambda i: (0, 0)),
            pl.BlockSpec(out_emb.shape, lambda i: (0, 0)),
        ],
        out_specs=pl.BlockSpec((BLK, EMB_DIM), lambda i: (i, 0)),
        out_shape=jax.ShapeDtypeStruct((N_NODES, EMB_DIM), jnp.float32),
    )(x, Wt, b2, deg, deg, in_emb, out_emb)


def kernel(x, edge_index, W, b, in_emb, out_emb):
    ei = edge_index.astype(jnp.int32)
    ei = jnp.pad(ei, ((0, 0), (0, E_PAD - E_TOTAL)))
    deg = _sc_degrees(ei).reshape(2, 1, N_PAD)
    return _tc_pass(x, W.T, b.reshape(1, EMB_DIM), deg,
                    in_emb.astype(jnp.bfloat16), out_emb.astype(jnp.bfloat16))
